# baseline (speedup 1.0000x reference)
import jax
import jax.numpy as jnp
from jax import lax
from jax.experimental import pallas as pl
from jax.experimental.pallas import tpu as pltpu

_SIZE = 16384
_INPUT = 4096
_K = 25
_BR = 1024
_NB = _SIZE // _BR
_DEPTH = 3


def _body(t_ref, perms_hbm, out_ref, buf, ov_ref, sems):
    i = pl.program_id(0)

    def copy(k, b):
        return pltpu.make_async_copy(
            perms_hbm.at[pl.ds(k * _BR, _BR)], buf.at[b], sems.at[b])

    @pl.when(i == 0)
    def _prime():
        copy(0, 0).start()
        copy(1, 1).start()

    @pl.when(i + 2 < _NB)
    def _prefetch():
        k = i + 2
        copy(k, lax.rem(k, _DEPTH)).start()

    b = lax.rem(i, _DEPTH)
    copy(i, b).wait()
    blk = buf[b]
    mask = (blk > t_ref[...]).astype(jnp.float32)
    ov_ref[i, :] = jnp.sum(mask, axis=1)

    @pl.when(i == _NB - 1)
    def _finish():
        ovs = ov_ref[...]

        def step(_, carry):
            lo, hi = carry
            mid = (lo + hi) // 2
            cnt = jnp.sum((ovs >= mid.astype(jnp.float32)).astype(jnp.int32))
            ok = cnt >= _K + 1
            return jnp.where(ok, mid, lo), jnp.where(ok, hi, mid)

        lo, _ = lax.fori_loop(
            0, 13, step, (jnp.int32(0), jnp.int32(_INPUT + 1)))
        out_ref[...] = (ovs > lo.astype(jnp.float32)).astype(jnp.int32)


def kernel(input_vector, perms):
    thresholds = jnp.where(input_vector, jnp.float32(0.6), jnp.inf)
    thresholds = thresholds.reshape(1, _INPUT)
    out = pl.pallas_call(
        _body,
        grid=(_NB,),
        in_specs=[
            pl.BlockSpec((1, _INPUT), lambda i: (0, 0)),
            pl.BlockSpec(memory_space=pl.ANY),
        ],
        out_specs=pl.BlockSpec((_NB, _BR), lambda i: (0, 0)),
        out_shape=jax.ShapeDtypeStruct((_NB, _BR), jnp.int32),
        scratch_shapes=[
            pltpu.VMEM((_DEPTH, _BR, _INPUT), jnp.float32),
            pltpu.VMEM((_NB, _BR), jnp.float32),
            pltpu.SemaphoreType.DMA((_DEPTH,)),
        ],
    )(thresholds, perms)
    return out.reshape(_SIZE).astype(jnp.bool_)
